# baseline (speedup 1.0000x reference)
import functools

import jax
import jax.numpy as jnp
from jax import lax
from jax.experimental import pallas as pl
from jax.experimental.pallas import tpu as pltpu
from jax.experimental.pallas import tpu_sc as plsc

PN = 8
D = 2048
HW = 128
RFD = 256
FLAT = D * HW
NSUB = 16
CHUNK = FLAT // NSUB
LANES = 16
STEPS = CHUNK // LANES
A_CONST = 1.0 / 32768.0

_SC_CACHE = {}


def _get_sc_partials():
    if "partials" not in _SC_CACHE:
        mesh = plsc.VectorSubcoreMesh(
            core_axis_name="c", subcore_axis_name="s", num_cores=1
        )
        _SC_CACHE["partials"] = functools.partial(
            pl.kernel,
            out_type=jax.ShapeDtypeStruct((NSUB, 16), jnp.float32),
            mesh=mesh,
            scratch_types=[
                pltpu.VMEM((CHUNK,), jnp.float32),
                pltpu.VMEM((CHUNK,), jnp.float32),
                pltpu.VMEM((16,), jnp.float32),
            ],
        )(_sc_partials_body)
    return _SC_CACHE["partials"]


def _get_sc_rank():
    if "rank" not in _SC_CACHE:
        mesh = plsc.VectorSubcoreMesh(
            core_axis_name="c", subcore_axis_name="s", num_cores=1
        )
        _SC_CACHE["rank"] = functools.partial(
            pl.kernel,
            out_type=jax.ShapeDtypeStruct((16,), jnp.int32),
            mesh=mesh,
            scratch_types=[
                pltpu.VMEM((NSUB, 16), jnp.float32),
                pltpu.VMEM((16,), jnp.int32),
            ],
        )(_sc_rank_body)
    return _SC_CACHE["rank"]


def _sc_partials_body(q_hbm, p_hbm, parts_out, qbuf, pbuf, part):
    wid = lax.axis_index("s")
    base = wid * CHUNK
    pltpu.sync_copy(q_hbm.at[pl.ds(base, CHUNK)], qbuf)
    lane = lax.iota(jnp.int32, 16)
    partv = jnp.zeros((16,), jnp.float32)
    for j in range(PN):
        pltpu.sync_copy(p_hbm.at[j, pl.ds(base, CHUNK)], pbuf)

        def body(i, acc):
            pv = pbuf[pl.ds(i * LANES, LANES)]
            qv = qbuf[pl.ds(i * LANES, LANES)]
            return acc + pv * (pv - 2.0 * qv)

        acc = lax.fori_loop(0, STEPS, body, jnp.zeros((16,), jnp.float32))
        sj = acc[0]
        for l in range(1, LANES):
            sj = sj + acc[l]
        partv = jnp.where(lane == j, sj, partv)
    part[...] = partv
    pltpu.sync_copy(part, parts_out.at[wid])


def _sc_rank_body(parts_hbm, idx_out, sbuf, ivm):
    wid = lax.axis_index("s")

    @pl.when(wid == 0)
    def _():
        pltpu.sync_copy(parts_hbm, sbuf)
        tot = jnp.zeros((16,), jnp.float32)
        for w in range(NSUB):
            tot = tot + sbuf[w]
        lane = lax.iota(jnp.int32, 16)
        valid = lane < PN
        idxv = jnp.zeros((16,), jnp.int32)
        for j in range(PN):
            sj = tot[j]
            lt = jnp.where((tot < sj) & valid, 1, 0)
            eq = jnp.where((tot == sj) & (lane < j), 1, 0)
            cnt = lt + eq
            r = cnt[0]
            for l in range(1, PN):
                r = r + cnt[l]
            idxv = idxv + jnp.where(lane == r, j, 0)
        ivm[...] = idxv
        pltpu.sync_copy(ivm, idx_out)


def _rf_body(rf_w_ref, p_ref, x_ref):
    x_ref[0] = jnp.dot(rf_w_ref[...], p_ref[0], preferred_element_type=jnp.float32)


def _main_body(idx_ref, x_ref, rf_mv_ref, rm_s_ref, fuse_ref, out_ref, xg_ref):
    for g in range(PN):
        xg_ref[g] = x_ref[idx_ref[g]]
    xg = xg_ref[...].reshape(D, HW)
    mvl = jnp.dot(rf_mv_ref[...], xg, preferred_element_type=jnp.float32)
    xfl = x_ref[...].reshape(D, HW)
    m2 = jnp.maximum(0.5 * xfl + (1.0 - A_CONST) * mvl, 0.0)
    t = jnp.dot(rm_s_ref[...], m2, preferred_element_type=jnp.float32)
    mu = jnp.mean(t, axis=1, keepdims=True)
    ctr = t - mu
    var = jnp.mean(ctr * ctr, axis=1, keepdims=True)
    tn = ctr * lax.rsqrt(var + 1e-5)
    z = jnp.dot(fuse_ref[...], tn, preferred_element_type=jnp.float32)
    out_ref[...] = jax.nn.sigmoid(z)


def _fuse_body(xf_ref, in_ref, o_ref):
    o_ref[...] = in_ref[...] * (1.0 + xf_ref[...])


def kernel(inputs, mv_prototype, rf_W, rf_mv_W, rm_s_W, fuse_W):
    n = inputs.shape[0]
    q = inputs[n - 1].reshape(FLAT)
    p_flat = mv_prototype.reshape(PN, FLAT)
    parts = _get_sc_partials()(q, p_flat)
    idx16 = _get_sc_rank()(parts)

    p3 = mv_prototype.reshape(PN, D, HW)
    x = pl.pallas_call(
        _rf_body,
        grid=(PN,),
        in_specs=[
            pl.BlockSpec((RFD, D), lambda p: (0, 0)),
            pl.BlockSpec((1, D, HW), lambda p: (p, 0, 0)),
        ],
        out_specs=pl.BlockSpec((1, RFD, HW), lambda p: (p, 0, 0)),
        out_shape=jax.ShapeDtypeStruct((PN, RFD, HW), jnp.float32),
    )(rf_W, p3)

    x_fuse = pl.pallas_call(
        _main_body,
        in_specs=[
            pl.BlockSpec(memory_space=pltpu.SMEM),
            pl.BlockSpec(memory_space=pltpu.VMEM),
            pl.BlockSpec(memory_space=pltpu.VMEM),
            pl.BlockSpec(memory_space=pltpu.VMEM),
            pl.BlockSpec(memory_space=pltpu.VMEM),
        ],
        out_specs=pl.BlockSpec(memory_space=pltpu.VMEM),
        out_shape=jax.ShapeDtypeStruct((D, HW), jnp.float32),
        scratch_shapes=[pltpu.VMEM((PN, RFD, HW), jnp.float32)],
    )(idx16, x, rf_mv_W, rm_s_W, fuse_W)

    inp3 = inputs.reshape(n, D, HW)
    bb = 4
    feats = pl.pallas_call(
        _fuse_body,
        grid=(n // bb,),
        in_specs=[
            pl.BlockSpec((D, HW), lambda i: (0, 0)),
            pl.BlockSpec((bb, D, HW), lambda i: (i, 0, 0)),
        ],
        out_specs=pl.BlockSpec((bb, D, HW), lambda i: (i, 0, 0)),
        out_shape=jax.ShapeDtypeStruct((n, D, HW), jnp.float32),
        compiler_params=pltpu.CompilerParams(
            dimension_semantics=("arbitrary",),
        ),
    )(x_fuse, inp3)
    return feats.reshape(inputs.shape)


# speedup vs baseline: 4.1831x; 4.1831x over previous
import functools

import jax
import jax.numpy as jnp
from jax import lax
from jax.experimental import pallas as pl
from jax.experimental.pallas import tpu as pltpu
from jax.experimental.pallas import tpu_sc as plsc

PN = 8
D = 2048
H = 16
W = 8
HW = H * W
RFD = 256
LANES = 16
A_CONST = 1.0 / 32768.0

_NT = (((1,), (1,)), ((), ()))

_SC_CACHE = {}


def _get_sc_rank():
    if "rank" not in _SC_CACHE:
        mesh = plsc.VectorSubcoreMesh(
            core_axis_name="c", subcore_axis_name="s", num_cores=1
        )
        _SC_CACHE["rank"] = functools.partial(
            pl.kernel,
            out_type=jax.ShapeDtypeStruct((16,), jnp.int32),
            mesh=mesh,
            scratch_types=[
                pltpu.VMEM((16,), jnp.float32),
                pltpu.VMEM((16,), jnp.int32),
            ],
        )(_sc_rank_body)
    return _SC_CACHE["rank"]


def _sc_rank_body(scores_hbm, idx_out, sbuf, ivm):
    wid = lax.axis_index("s")

    @pl.when(wid == 0)
    def _():
        pltpu.sync_copy(scores_hbm, sbuf)
        tot = sbuf[...]
        lane = lax.iota(jnp.int32, 16)
        valid = lane < PN
        idxv = jnp.zeros((16,), jnp.int32)
        for j in range(PN):
            sj = tot[j]
            lt = jnp.where((tot < sj) & valid, 1, 0)
            eq = jnp.where((tot == sj) & (lane < j), 1, 0)
            cnt = lt + eq
            r = cnt[0]
            for l in range(1, PN):
                r = r + cnt[l]
            idxv = idxv + jnp.where(lane == r, j, 0)
        ivm[...] = idxv
        pltpu.sync_copy(ivm, idx_out)


def _rf_body(q_ref, rf_w_ref, p_ref, x_ref, s_ref):
    p = pl.program_id(0)
    pm = p_ref[0]
    x_ref[0] = lax.dot_general(pm, rf_w_ref[...], _NT,
                               preferred_element_type=jnp.float32)
    q = q_ref[...]
    s_ref[p] = jnp.sum(pm * (pm - 2.0 * q))


def _main_body(idx_ref, x_ref, rf_mv_ref, rm_s_ref, fuse_ref, out_ref):
    acc = jnp.zeros((HW, D), jnp.float32)
    for g in range(PN):
        xg = x_ref[idx_ref[g]]
        wg = rf_mv_ref[:, g * RFD:(g + 1) * RFD]
        acc = acc + lax.dot_general(xg, wg, _NT,
                                    preferred_element_type=jnp.float32)
    xcat = jnp.concatenate([x_ref[g] for g in range(PN)], axis=1)
    m2 = jnp.maximum(0.5 * xcat + (1.0 - A_CONST) * acc, 0.0)
    t = lax.dot_general(m2, rm_s_ref[...], _NT,
                        preferred_element_type=jnp.float32)
    mu = jnp.mean(t, axis=0, keepdims=True)
    ctr = t - mu
    var = jnp.mean(ctr * ctr, axis=0, keepdims=True)
    tn = ctr * lax.rsqrt(var + 1e-5)
    z = lax.dot_general(tn, fuse_ref[...], _NT,
                        preferred_element_type=jnp.float32)
    out_ref[...] = jax.nn.sigmoid(z)


def _fuse_body(xf_ref, in_ref, o_ref):
    o_ref[...] = in_ref[...] * (1.0 + xf_ref[...])


def kernel(inputs, mv_prototype, rf_W, rf_mv_W, rm_s_W, fuse_W):
    n = inputs.shape[0]
    pin = jnp.transpose(inputs, (0, 2, 3, 1)).reshape(n, HW, D)
    pT = jnp.transpose(mv_prototype, (0, 2, 3, 1)).reshape(PN, HW, D)
    qT = pin[n - 1]

    xT, scores16 = pl.pallas_call(
        _rf_body,
        grid=(PN,),
        in_specs=[
            pl.BlockSpec((HW, D), lambda p: (0, 0)),
            pl.BlockSpec((RFD, D), lambda p: (0, 0)),
            pl.BlockSpec((1, HW, D), lambda p: (p, 0, 0)),
        ],
        out_specs=[
            pl.BlockSpec((1, HW, RFD), lambda p: (p, 0, 0)),
            pl.BlockSpec(memory_space=pltpu.SMEM),
        ],
        out_shape=[
            jax.ShapeDtypeStruct((PN, HW, RFD), jnp.float32),
            jax.ShapeDtypeStruct((16,), jnp.float32),
        ],
        compiler_params=pltpu.CompilerParams(
            dimension_semantics=("arbitrary",),
        ),
    )(qT, rf_W, pT)

    idx16 = _get_sc_rank()(scores16)

    xfT = pl.pallas_call(
        _main_body,
        in_specs=[
            pl.BlockSpec(memory_space=pltpu.SMEM),
            pl.BlockSpec(memory_space=pltpu.VMEM),
            pl.BlockSpec(memory_space=pltpu.VMEM),
            pl.BlockSpec(memory_space=pltpu.VMEM),
            pl.BlockSpec(memory_space=pltpu.VMEM),
        ],
        out_specs=pl.BlockSpec(memory_space=pltpu.VMEM),
        out_shape=jax.ShapeDtypeStruct((HW, D), jnp.float32),
    )(idx16, xT, rf_mv_W, rm_s_W, fuse_W)

    bb = 4
    out = pl.pallas_call(
        _fuse_body,
        grid=(n // bb,),
        in_specs=[
            pl.BlockSpec((HW, D), lambda i: (0, 0)),
            pl.BlockSpec((bb, HW, D), lambda i: (i, 0, 0)),
        ],
        out_specs=pl.BlockSpec((bb, HW, D), lambda i: (i, 0, 0)),
        out_shape=jax.ShapeDtypeStruct((n, HW, D), jnp.float32),
        compiler_params=pltpu.CompilerParams(
            dimension_semantics=("arbitrary",),
        ),
    )(xfT, pin)

    return jnp.transpose(out.reshape(n, H, W, D), (0, 3, 1, 2))
